# baseline (speedup 1.0000x reference)
import functools

import jax
import jax.numpy as jnp
from jax import lax
from jax.experimental import pallas as pl
from jax.experimental.pallas import tpu as pltpu
from jax.experimental.pallas import tpu_sc as plsc

_N = 64
_NC = 2
_NS = 16
_NW = _NC * _NS
_CH = 4


def _bcast_lane(vec, l):
    idx = jnp.full((16, 1), l, jnp.int32)
    dn = lax.GatherDimensionNumbers(
        offset_dims=(), collapsed_slice_dims=(0,), start_index_map=(0,)
    )
    return lax.gather(
        vec, idx, dn, slice_sizes=(1,),
        mode=lax.GatherScatterMode.PROMISE_IN_BOUNDS,
    )


def _ostore(obuf, p, i, k, v):
    w = 4 * i + k
    obuf[p, w // 8, (w % 8) * 16 : (w % 8) * 16 + 16] = v


def _emit_pair(xbuf, obuf, p, lp, lane):
    pvec = jnp.full((16,), lp // 2, jnp.int32)
    rem = (lp % 2) * 64
    xv = [
        plsc.load_gather(xbuf, (pvec, lane + (rem + 16 * k))) for k in range(4)
    ]
    r = [jnp.zeros((16,), jnp.float32)] * 4
    for i in range(_N - 1, -1, -1):
        q, li = divmod(i, 16)
        b = _bcast_lane(xv[q], li)
        r[q] = jnp.where(lane > li, jnp.maximum(r[q], b), b)
        for k in range(q + 1, 4):
            r[k] = jnp.maximum(r[k], b)
        _ostore(
            obuf, p, i, q,
            jnp.where(lane >= li, r[q], jnp.zeros((16,), jnp.float32)),
        )
        for k in range(q + 1, 4):
            _ostore(obuf, p, i, k, r[k])


def _prezero_lower(obuf):
    zeros = jnp.zeros((16,), jnp.float32)
    for p in range(_CH):
        for i in range(_N):
            for k in range(i // 16):
                _ostore(obuf, p, i, k, zeros)


def _write_mask(obuf, lane):
    zeros = jnp.zeros((16,), jnp.float32)
    ones = jnp.ones((16,), jnp.float32)
    for i in range(_N):
        q, li = divmod(i, 16)
        _ostore(obuf, 0, i, q, jnp.where(lane >= li, ones, zeros))
        for k in range(q + 1, 4):
            _ostore(obuf, 0, i, k, ones)


def kernel(x):
    B, H, N = x.shape
    P = B * H
    per_w = P // _NW
    n_half = per_w // _CH // 2
    xf = x.reshape(P // 2, 2 * N)
    mesh = plsc.VectorSubcoreMesh(
        core_axis_name="c", subcore_axis_name="s", num_cores=_NC, num_subcores=_NS
    )

    @functools.partial(
        pl.kernel,
        out_type=[
            jax.ShapeDtypeStruct((P, N * N // 128, 128), x.dtype),
            jax.ShapeDtypeStruct((B, N * N // 128, 128), x.dtype),
        ],
        mesh=mesh,
        compiler_params=pltpu.CompilerParams(needs_layout_passes=False),
        scratch_types=[
            pltpu.VMEM((per_w // 2, 2 * N), jnp.float32),
            pltpu.VMEM((_CH, N * N // 128, 128), jnp.float32),
            pltpu.VMEM((_CH, N * N // 128, 128), jnp.float32),
            pltpu.SemaphoreType.DMA,
            pltpu.SemaphoreType.DMA,
        ],
    )
    def sc_kernel(x_hbm, out_hbm, mask_hbm, xbuf, obuf0, obuf1, sem0, sem1):
        cid = lax.axis_index("c")
        sid = lax.axis_index("s")
        wid = sid * _NC + cid
        base = wid * per_w
        lane = lax.iota(jnp.int32, 16)

        xstart = pl.multiple_of(base // 2, per_w // 2)
        pltpu.sync_copy(x_hbm.at[pl.ds(xstart, per_w // 2)], xbuf)
        _prezero_lower(obuf0)
        _prezero_lower(obuf1)

        _write_mask(obuf0, lane)
        pltpu.sync_copy(obuf0.at[pl.ds(0, 1)], mask_hbm.at[pl.ds(wid, 1)])

        def body(cc, carry):
            ci0 = 2 * cc
            ci1 = 2 * cc + 1
            dst0 = out_hbm.at[pl.ds(base + ci0 * _CH, _CH)]
            dst1 = out_hbm.at[pl.ds(base + ci1 * _CH, _CH)]

            @pl.when(cc > 0)
            def _():
                pltpu.make_async_copy(obuf0, dst0, sem0).wait()

            for p in range(_CH):
                _emit_pair(xbuf, obuf0, p, ci0 * _CH + p, lane)
            pltpu.async_copy(obuf0, dst0, sem0)

            @pl.when(cc > 0)
            def _():
                pltpu.make_async_copy(obuf1, dst1, sem1).wait()

            for p in range(_CH):
                _emit_pair(xbuf, obuf1, p, ci1 * _CH + p, lane)
            pltpu.async_copy(obuf1, dst1, sem1)
            return carry

        lax.fori_loop(0, n_half, body, 0)
        pltpu.make_async_copy(obuf0, out_hbm.at[pl.ds(base, _CH)], sem0).wait()
        pltpu.make_async_copy(obuf1, out_hbm.at[pl.ds(base, _CH)], sem1).wait()

    out_flat, out_mask = sc_kernel(xf)
    return out_flat.reshape(B, H, N, N), out_mask.reshape(B, 1, N, N)


# speedup vs baseline: 1.7380x; 1.7380x over previous
import functools

import jax
import jax.numpy as jnp
from jax import lax
from jax.experimental import pallas as pl
from jax.experimental.pallas import tpu as pltpu
from jax.experimental.pallas import tpu_sc as plsc

_N = 64
_NC = 2
_NS = 16
_NW = _NC * _NS
_CH = 4


def _bcast_lane(vec, l):
    idx = jnp.full((16, 1), l, jnp.int32)
    dn = lax.GatherDimensionNumbers(
        offset_dims=(), collapsed_slice_dims=(0,), start_index_map=(0,)
    )
    return lax.gather(
        vec, idx, dn, slice_sizes=(1,),
        mode=lax.GatherScatterMode.PROMISE_IN_BOUNDS,
    )


def _ostore(obuf, p, i, k, v):
    obuf[p, i, k * 16 : k * 16 + 16] = v


def _emit_pair(xbuf, obuf, p, lp, lane):
    pvec = jnp.full((16,), lp // 2, jnp.int32)
    rem = (lp % 2) * 64
    xv = [
        plsc.load_gather(xbuf, (pvec, lane + (rem + 16 * k))) for k in range(4)
    ]
    r = [jnp.zeros((16,), jnp.float32)] * 4
    for i in range(_N - 1, -1, -1):
        q, li = divmod(i, 16)
        b = _bcast_lane(xv[q], li)
        r[q] = jnp.where(lane > li, jnp.maximum(r[q], b), b)
        for k in range(q + 1, 4):
            r[k] = jnp.maximum(r[k], b)
        _ostore(
            obuf, p, i, q,
            jnp.where(lane >= li, r[q], jnp.zeros((16,), jnp.float32)),
        )
        for k in range(q + 1, 4):
            _ostore(obuf, p, i, k, r[k])


def _prezero_lower(obuf):
    zeros = jnp.zeros((16,), jnp.float32)
    for p in range(_CH):
        for i in range(_N):
            for k in range(i // 16):
                _ostore(obuf, p, i, k, zeros)


def _write_mask(obuf, lane):
    zeros = jnp.zeros((16,), jnp.float32)
    ones = jnp.ones((16,), jnp.float32)
    for i in range(_N):
        q, li = divmod(i, 16)
        _ostore(obuf, 0, i, q, jnp.where(lane >= li, ones, zeros))
        for k in range(q + 1, 4):
            _ostore(obuf, 0, i, k, ones)


def kernel(x):
    B, H, N = x.shape
    P = B * H
    per_w = P // _NW
    n_half = per_w // _CH // 2
    xf = x.reshape(P // 2, 2 * N)
    mesh = plsc.VectorSubcoreMesh(
        core_axis_name="c", subcore_axis_name="s", num_cores=_NC, num_subcores=_NS
    )

    @functools.partial(
        pl.kernel,
        out_type=[
            jax.ShapeDtypeStruct((P, N, N), x.dtype),
            jax.ShapeDtypeStruct((B, N, N), x.dtype),
        ],
        mesh=mesh,
        compiler_params=pltpu.CompilerParams(needs_layout_passes=False),
        scratch_types=[
            pltpu.VMEM((per_w // 2, 2 * N), jnp.float32),
            pltpu.VMEM((_CH, N, N), jnp.float32),
            pltpu.VMEM((_CH, N, N), jnp.float32),
            pltpu.SemaphoreType.DMA,
            pltpu.SemaphoreType.DMA,
        ],
    )
    def sc_kernel(x_hbm, out_hbm, mask_hbm, xbuf, obuf0, obuf1, sem0, sem1):
        cid = lax.axis_index("c")
        sid = lax.axis_index("s")
        wid = sid * _NC + cid
        base = wid * per_w
        lane = lax.iota(jnp.int32, 16)

        xstart = pl.multiple_of(base // 2, per_w // 2)
        pltpu.sync_copy(x_hbm.at[pl.ds(xstart, per_w // 2)], xbuf)
        _prezero_lower(obuf0)
        _prezero_lower(obuf1)

        _write_mask(obuf0, lane)
        pltpu.sync_copy(obuf0.at[pl.ds(0, 1)], mask_hbm.at[pl.ds(wid, 1)])

        def body(cc, carry):
            ci0 = 2 * cc
            ci1 = 2 * cc + 1
            dst0 = out_hbm.at[pl.ds(base + ci0 * _CH, _CH)]
            dst1 = out_hbm.at[pl.ds(base + ci1 * _CH, _CH)]

            @pl.when(cc > 0)
            def _():
                pltpu.make_async_copy(obuf0, dst0, sem0).wait()

            for p in range(_CH):
                _emit_pair(xbuf, obuf0, p, ci0 * _CH + p, lane)
            pltpu.async_copy(obuf0, dst0, sem0)

            @pl.when(cc > 0)
            def _():
                pltpu.make_async_copy(obuf1, dst1, sem1).wait()

            for p in range(_CH):
                _emit_pair(xbuf, obuf1, p, ci1 * _CH + p, lane)
            pltpu.async_copy(obuf1, dst1, sem1)
            return carry

        lax.fori_loop(0, n_half, body, 0)
        pltpu.make_async_copy(obuf0, out_hbm.at[pl.ds(base, _CH)], sem0).wait()
        pltpu.make_async_copy(obuf1, out_hbm.at[pl.ds(base, _CH)], sem1).wait()

    out_flat, out_mask = sc_kernel(xf)
    return out_flat.reshape(B, H, N, N), out_mask.reshape(B, 1, N, N)


# speedup vs baseline: 1.7400x; 1.0011x over previous
import functools

import jax
import jax.numpy as jnp
from jax import lax
from jax.experimental import pallas as pl
from jax.experimental.pallas import tpu as pltpu
from jax.experimental.pallas import tpu_sc as plsc

_N = 64
_NC = 2
_NS = 16
_NW = _NC * _NS
_CH = 2
_NB = 4


def _bcast_lane(vec, l):
    idx = jnp.full((16, 1), l, jnp.int32)
    dn = lax.GatherDimensionNumbers(
        offset_dims=(), collapsed_slice_dims=(0,), start_index_map=(0,)
    )
    return lax.gather(
        vec, idx, dn, slice_sizes=(1,),
        mode=lax.GatherScatterMode.PROMISE_IN_BOUNDS,
    )


def _ostore(obuf, p, i, k, v):
    obuf[p, i, k * 16 : k * 16 + 16] = v


def _emit_pair(xbuf, obuf, p, lp, lane):
    pvec = jnp.full((16,), lp // 2, jnp.int32)
    rem = (lp % 2) * 64
    xv = [
        plsc.load_gather(xbuf, (pvec, lane + (rem + 16 * k))) for k in range(4)
    ]
    r = [jnp.zeros((16,), jnp.float32)] * 4
    for i in range(_N - 1, -1, -1):
        q, li = divmod(i, 16)
        b = _bcast_lane(xv[q], li)
        r[q] = jnp.where(lane > li, jnp.maximum(r[q], b), b)
        for k in range(q + 1, 4):
            r[k] = jnp.maximum(r[k], b)
        _ostore(
            obuf, p, i, q,
            jnp.where(lane >= li, r[q], jnp.zeros((16,), jnp.float32)),
        )
        for k in range(q + 1, 4):
            _ostore(obuf, p, i, k, r[k])


def _prezero_lower(obuf):
    zeros = jnp.zeros((16,), jnp.float32)
    for p in range(_CH):
        for i in range(_N):
            for k in range(i // 16):
                _ostore(obuf, p, i, k, zeros)


def _write_mask(obuf, lane):
    zeros = jnp.zeros((16,), jnp.float32)
    ones = jnp.ones((16,), jnp.float32)
    for i in range(_N):
        q, li = divmod(i, 16)
        _ostore(obuf, 0, i, q, jnp.where(lane >= li, ones, zeros))
        for k in range(q + 1, 4):
            _ostore(obuf, 0, i, k, ones)


def kernel(x):
    B, H, N = x.shape
    P = B * H
    per_w = P // _NW
    n_ring = per_w // _CH // _NB
    xf = x.reshape(P // 2, 2 * N)
    mesh = plsc.VectorSubcoreMesh(
        core_axis_name="c", subcore_axis_name="s", num_cores=_NC, num_subcores=_NS
    )

    @functools.partial(
        pl.kernel,
        out_type=[
            jax.ShapeDtypeStruct((P, N, N), x.dtype),
            jax.ShapeDtypeStruct((B, N, N), x.dtype),
        ],
        mesh=mesh,
        compiler_params=pltpu.CompilerParams(needs_layout_passes=False),
        scratch_types=[
            pltpu.VMEM((per_w // 2, 2 * N), jnp.float32),
        ] + [pltpu.VMEM((_CH, N, N), jnp.float32) for _ in range(_NB)]
        + [pltpu.SemaphoreType.DMA for _ in range(_NB)],
    )
    def sc_kernel(x_hbm, out_hbm, mask_hbm, xbuf, *bufs_and_sems):
        obufs = bufs_and_sems[:_NB]
        sems = bufs_and_sems[_NB:]
        cid = lax.axis_index("c")
        sid = lax.axis_index("s")
        wid = sid * _NC + cid
        base = wid * per_w
        lane = lax.iota(jnp.int32, 16)

        xstart = pl.multiple_of(base // 2, per_w // 2)
        pltpu.sync_copy(x_hbm.at[pl.ds(xstart, per_w // 2)], xbuf)
        for ob in obufs:
            _prezero_lower(ob)

        _write_mask(obufs[0], lane)
        pltpu.sync_copy(obufs[0].at[pl.ds(0, 1)], mask_hbm.at[pl.ds(wid, 1)])

        def body(cc, carry):
            for s in range(_NB):
                ci = _NB * cc + s
                dst = out_hbm.at[pl.ds(base + ci * _CH, _CH)]

                @pl.when(cc > 0)
                def _():
                    pltpu.make_async_copy(obufs[s], dst, sems[s]).wait()

                for p in range(_CH):
                    _emit_pair(xbuf, obufs[s], p, ci * _CH + p, lane)
                pltpu.async_copy(obufs[s], dst, sems[s])
            return carry

        lax.fori_loop(0, n_ring, body, 0)
        for s in range(_NB):
            pltpu.make_async_copy(
                obufs[s], out_hbm.at[pl.ds(base, _CH)], sems[s]
            ).wait()

    out_flat, out_mask = sc_kernel(xf)
    return out_flat.reshape(B, H, N, N), out_mask.reshape(B, 1, N, N)


# speedup vs baseline: 1.7697x; 1.0170x over previous
import functools

import jax
import jax.numpy as jnp
from jax import lax
from jax.experimental import pallas as pl
from jax.experimental.pallas import tpu as pltpu
from jax.experimental.pallas import tpu_sc as plsc

_N = 64
_NC = 2
_NS = 16
_NW = _NC * _NS
_CH = 2


def _bcast_lane(vec, l):
    idx = jnp.full((16, 1), l, jnp.int32)
    dn = lax.GatherDimensionNumbers(
        offset_dims=(), collapsed_slice_dims=(0,), start_index_map=(0,)
    )
    return lax.gather(
        vec, idx, dn, slice_sizes=(1,),
        mode=lax.GatherScatterMode.PROMISE_IN_BOUNDS,
    )


def _emit_pair(xbuf, obuf, p, lp, lane):
    pvec = jnp.full((16,), lp, jnp.int32)
    xv = [
        plsc.load_gather(xbuf, (pvec, lane + 16 * k)) for k in range(4)
    ]
    r = [jnp.zeros((16,), jnp.float32)] * 4
    for i in range(_N - 1, -1, -1):
        q, li = divmod(i, 16)
        b = _bcast_lane(xv[q], li)
        r[q] = jnp.where(lane > li, jnp.maximum(r[q], b), b)
        for k in range(q + 1, 4):
            r[k] = jnp.maximum(r[k], b)
        obuf[p, i, q * 16 : (q + 1) * 16] = jnp.where(
            lane >= li, r[q], jnp.zeros((16,), jnp.float32)
        )
        for k in range(q + 1, 4):
            obuf[p, i, k * 16 : (k + 1) * 16] = r[k]


def _prezero_lower(obuf):
    zeros = jnp.zeros((16,), jnp.float32)
    for p in range(_CH):
        for i in range(_N):
            for k in range(i // 16):
                obuf[p, i, k * 16 : (k + 1) * 16] = zeros


def _write_mask(mbuf, lane):
    zeros = jnp.zeros((16,), jnp.float32)
    ones = jnp.ones((16,), jnp.float32)
    for i in range(_N):
        q, li = divmod(i, 16)
        for k in range(q):
            mbuf[0, 0, i, k * 16 : (k + 1) * 16] = zeros
        mbuf[0, 0, i, q * 16 : (q + 1) * 16] = jnp.where(lane >= li, ones, zeros)
        for k in range(q + 1, 4):
            mbuf[0, 0, i, k * 16 : (k + 1) * 16] = ones


def kernel(x):
    B, H, N = x.shape
    P = B * H
    per_w = P // _NW
    n_half = per_w // _CH // 2
    xf = x.reshape(P, N)
    mesh = plsc.VectorSubcoreMesh(
        core_axis_name="c", subcore_axis_name="s", num_cores=_NC, num_subcores=_NS
    )

    @functools.partial(
        pl.kernel,
        out_type=[
            jax.ShapeDtypeStruct((P, N, N), x.dtype),
            jax.ShapeDtypeStruct((B, 1, N, N), x.dtype),
        ],
        mesh=mesh,
        compiler_params=pltpu.CompilerParams(needs_layout_passes=False),
        scratch_types=[
            pltpu.VMEM((per_w, N), jnp.float32),
            pltpu.VMEM((_CH, N, N), jnp.float32),
            pltpu.VMEM((_CH, N, N), jnp.float32),
            pltpu.VMEM((1, 1, N, N), jnp.float32),
            pltpu.SemaphoreType.DMA,
            pltpu.SemaphoreType.DMA,
        ],
    )
    def sc_kernel(x_hbm, out_hbm, mask_hbm, xbuf, obuf0, obuf1, mbuf, sem0, sem1):
        cid = lax.axis_index("c")
        sid = lax.axis_index("s")
        wid = sid * _NC + cid
        base = wid * per_w
        lane = lax.iota(jnp.int32, 16)

        pltpu.sync_copy(x_hbm.at[pl.ds(base, per_w)], xbuf)
        _prezero_lower(obuf0)
        _prezero_lower(obuf1)

        _write_mask(mbuf, lane)
        pltpu.sync_copy(mbuf, mask_hbm.at[pl.ds(wid, 1)])

        def body(cc, carry):
            ci0 = 2 * cc
            ci1 = 2 * cc + 1
            dst0 = out_hbm.at[pl.ds(base + ci0 * _CH, _CH)]
            dst1 = out_hbm.at[pl.ds(base + ci1 * _CH, _CH)]

            @pl.when(cc > 0)
            def _():
                pltpu.make_async_copy(obuf0, dst0, sem0).wait()

            for p in range(_CH):
                _emit_pair(xbuf, obuf0, p, ci0 * _CH + p, lane)
            pltpu.async_copy(obuf0, dst0, sem0)

            @pl.when(cc > 0)
            def _():
                pltpu.make_async_copy(obuf1, dst1, sem1).wait()

            for p in range(_CH):
                _emit_pair(xbuf, obuf1, p, ci1 * _CH + p, lane)
            pltpu.async_copy(obuf1, dst1, sem1)
            return carry

        lax.fori_loop(0, n_half, body, 0)
        pltpu.make_async_copy(obuf0, out_hbm.at[pl.ds(base, _CH)], sem0).wait()
        pltpu.make_async_copy(obuf1, out_hbm.at[pl.ds(base, _CH)], sem1).wait()

    out_flat, out_mask = sc_kernel(xf)
    return out_flat.reshape(B, H, N, N), out_mask
